# baseline (speedup 1.0000x reference)
import jax
import jax.numpy as jnp
from jax.experimental import pallas as pl
from jax.experimental.pallas import tpu as pltpu

_BLOCK_COLS = 119808


def _pointwise_mm_block(xt_ref, w_ref, b_ref, ot_ref):
    ot_ref[...] = (
        jax.lax.dot_general(
            w_ref[...], xt_ref[...],
            dimension_numbers=(((0,), (0,)), ((), ())),
            preferred_element_type=jnp.float32,
        )
        + b_ref[...].T
    )


def kernel(input, kernel, bias):
    n, c_in = input.shape
    c_out = kernel.shape[1]
    xt = input.T
    grid = (pl.cdiv(n, _BLOCK_COLS),)
    out_t = pl.pallas_call(
        _pointwise_mm_block,
        grid=grid,
        in_specs=[
            pl.BlockSpec((c_in, _BLOCK_COLS), lambda i: (0, i)),
            pl.BlockSpec((c_in, c_out), lambda i: (0, 0)),
            pl.BlockSpec((1, c_out), lambda i: (0, 0)),
        ],
        out_specs=pl.BlockSpec((c_out, _BLOCK_COLS), lambda i: (0, i)),
        out_shape=jax.ShapeDtypeStruct((c_out, n), jnp.float32),
        compiler_params=pltpu.CompilerParams(
            dimension_semantics=("parallel",),
        ),
    )(xt, kernel, bias)
    return out_t.T
